# baseline (speedup 1.0000x reference)
import jax
import jax.numpy as jnp
from jax.experimental import pallas as pl
from jax.experimental.pallas import tpu as pltpu

S = 2048
RB = 128
CB = 256
N_CC = S // CB


def _body(clo_ref, chi_ref, cv_ref, x_ref, o_ref):
    rb = pl.program_id(0)
    cc = pl.program_id(1)
    skipped = (cc >= clo_ref[rb]) & (cc <= chi_ref[rb])

    @pl.when(skipped)
    def _():
        o_ref[...] = jnp.zeros_like(o_ref)

    @pl.when(jnp.logical_not(skipped))
    def _():
        cv = cv_ref[0]
        g = rb * RB + jax.lax.broadcasted_iota(jnp.int32, (RB, CB), 0)
        r = jax.lax.rem(g, S)
        i = jnp.minimum(r, S - 1 - r)
        val = jnp.clip((i.astype(jnp.float32) - 991.0 + 2048.0 * cv)
                       * (1.0 / 32.0), 0.0, 1.0)
        c = cc * CB + jax.lax.broadcasted_iota(jnp.int32, (RB, CB), 1)
        left = i + jnp.where(r >= S // 2, 1, 0)
        cond = (c >= left) & (c <= S - 1 - i)
        o_ref[...] = x_ref[...] * jnp.where(cond, val, 1.0)


def _skip_ranges(cv, n_rb):
    rb = jnp.arange(n_rb, dtype=jnp.int32)
    r0 = (rb * RB) % S
    top = (r0 + RB) <= (S // 2)
    max_i = jnp.where(top, r0 + RB - 1, S - 1 - r0)
    v = (max_i.astype(jnp.float32) - 991.0 + 2048.0 * cv[0])
    val0 = v <= 0.0
    max_left = max_i + jnp.where(top, 0, 1)
    min_right = S - 1 - max_i
    clo = (max_left + CB - 1) // CB
    chi = (min_right - (CB - 1)) // CB
    ok = val0 & (clo <= chi)
    clo = jnp.where(ok, clo, 1).astype(jnp.int32)
    chi = jnp.where(ok, chi, 0).astype(jnp.int32)
    return clo, chi


def kernel(x, current_val):
    B, H, Sr, Sc = x.shape
    x2 = x.reshape(B * H * Sr, Sc)
    n_rows = x2.shape[0]
    n_rb = n_rows // RB
    clo, chi = _skip_ranges(current_val, n_rb)

    def x_map(rb, cc, clo_ref, chi_ref):
        return rb, cc

    def o_map(rb, cc, clo_ref, chi_ref):
        return rb, cc

    grid_spec = pltpu.PrefetchScalarGridSpec(
        num_scalar_prefetch=2,
        grid=(n_rb, N_CC),
        in_specs=[
            pl.BlockSpec(memory_space=pltpu.SMEM),
            pl.BlockSpec((RB, CB), x_map),
        ],
        out_specs=pl.BlockSpec((RB, CB), o_map),
    )
    out = pl.pallas_call(
        _body,
        grid_spec=grid_spec,
        out_shape=jax.ShapeDtypeStruct((n_rows, Sc), x.dtype),
    )(clo, chi, current_val, x2)
    return out.reshape(B, H, Sr, Sc)


# speedup vs baseline: 1.7894x; 1.7894x over previous
import functools

import jax
import jax.numpy as jnp
from jax import lax
from jax.experimental import pallas as pl
from jax.experimental.pallas import tpu as pltpu
from jax.experimental.pallas import tpu_sc as plsc

S = 2048
ROWS = 12 * 2048
NC, NS, L = 2, 16, 16
NW = NC * NS
RPW = ROWS // NW
CH = 8
NCHUNK = RPW // CH
NB = 4
PF = 2


def _scale_row(bufs, b, j, row, cv):
    r = lax.rem(row, S)
    i = jnp.minimum(r, S - 1 - r)
    left = i + jnp.where(r >= S // 2, 1, 0)
    right = S - 1 - i
    val = jnp.clip((i.astype(jnp.float32) - 991.0 + 2048.0 * cv)
                   * (1.0 / 32.0), 0.0, 1.0)
    gl = left // L
    gr = right // L
    lane = lax.iota(jnp.int32, L)

    def edge(g):
        base = g * L
        c = base + lane
        v = bufs[b, j, pl.ds(base, L)]
        m = (c >= left) & (c <= right)
        bufs[b, j, pl.ds(base, L)] = jnp.where(m, v * val, v)

    @pl.when(val < 1.0)
    def _():
        edge(gl)

        @pl.when(gr != gl)
        def _():
            edge(gr)

        def inner(g, carry):
            base = g * L
            bufs[b, j, pl.ds(base, L)] = bufs[b, j, pl.ds(base, L)] * val
            return carry

        lax.fori_loop(gl + 1, gr, inner, 0)


def _sc_body(x_hbm, cv_hbm, o_hbm, cvv, bufs, lsem, ssem):
    cid = lax.axis_index("c")
    sid = lax.axis_index("s")
    wid = sid * NC + cid
    pltpu.sync_copy(cv_hbm, cvv)
    cv = cvv[pl.ds(0, L)][0]
    row0 = wid * RPW

    def load(c, b):
        pltpu.async_copy(x_hbm.at[pl.ds(row0 + c * CH, CH)],
                         bufs.at[b], lsem.at[b])

    def wait_load(b):
        pltpu.make_async_copy(x_hbm.at[pl.ds(0, CH)],
                              bufs.at[b], lsem.at[b]).wait()

    def store(c, b):
        pltpu.async_copy(bufs.at[b],
                         o_hbm.at[pl.ds(row0 + c * CH, CH)], ssem.at[b])

    def wait_store(b):
        pltpu.make_async_copy(bufs.at[b],
                              o_hbm.at[pl.ds(0, CH)], ssem.at[b]).wait()

    for c0 in range(PF):
        load(c0, c0 % NB)

    def outer(t, carry):
        c_base = t * NB
        for b in range(NB):
            c = c_base + b

            @pl.when(c + PF < NCHUNK)
            def _(c=c, b=b):
                @pl.when(c + PF - NB >= 0)
                def _():
                    wait_store((b + PF) % NB)
                load(c + PF, (b + PF) % NB)

            wait_load(b)
            for j in range(CH):
                _scale_row(bufs, b, j, row0 + c * CH + j, cv)
            store(c, b)
        return carry

    lax.fori_loop(0, NCHUNK // NB, outer, 0)

    for k in range(NB):
        wait_store((NCHUNK - NB + k) % NB)


def kernel(x, current_val):
    B, H, Sr, Sc = x.shape
    x2 = x.reshape(B * H * Sr, Sc)
    cvp = jnp.pad(current_val.astype(jnp.float32), (0, L - 1))
    mesh = plsc.VectorSubcoreMesh(core_axis_name="c", subcore_axis_name="s")
    k = functools.partial(
        pl.kernel,
        mesh=mesh,
        out_type=jax.ShapeDtypeStruct((ROWS, S), jnp.float32),
        scratch_types=[
            pltpu.VMEM((L,), jnp.float32),
            pltpu.VMEM((NB, CH, S), jnp.float32),
            pltpu.SemaphoreType.DMA((NB,)),
            pltpu.SemaphoreType.DMA((NB,)),
        ],
    )(_sc_body)
    out = k(x2, cvp)
    return out.reshape(B, H, Sr, Sc)


# speedup vs baseline: 3.6780x; 2.0555x over previous
import functools

import jax
import jax.numpy as jnp
from jax import lax
from jax.experimental import pallas as pl
from jax.experimental.pallas import tpu as pltpu
from jax.experimental.pallas import tpu_sc as plsc

S = 2048
ROWS = 12 * 2048
NC, NS, L = 2, 16, 16
NW = NC * NS
RPW = ROWS // NW
CH = 8
NCHUNK = RPW // CH
NB = 4
PF = 2


def _scale_row(bufs, b, j, row, cv):
    r = lax.rem(row, S)
    i = jnp.minimum(r, S - 1 - r)
    left = i + jnp.where(r >= S // 2, 1, 0)
    right = S - 1 - i
    val = jnp.clip((i.astype(jnp.float32) - 991.0 + 2048.0 * cv)
                   * (1.0 / 32.0), 0.0, 1.0)
    gl = left // L
    gr = right // L
    lane = lax.iota(jnp.int32, L)

    def edge(g):
        base = g * L
        c = base + lane
        v = bufs[b, j, pl.ds(base, L)]
        m = (c >= left) & (c <= right)
        bufs[b, j, pl.ds(base, L)] = jnp.where(m, v * val, v)

    @pl.when(val < 1.0)
    def _():
        edge(gl)

        @pl.when(gr != gl)
        def _():
            edge(gr)

        g0 = gl + 1
        n_int = jnp.maximum(gr - g0, 0)
        n4 = n_int // 4

        @pl.when(val <= 0.0)
        def _():
            zvec = jnp.zeros((L,), jnp.float32)

            def z4(k, carry):
                base = (g0 + 4 * k) * L
                for off in range(4):
                    bufs[b, j, pl.ds(base + off * L, L)] = zvec
                return carry

            lax.fori_loop(0, n4, z4, 0)

            def z1(g, carry):
                bufs[b, j, pl.ds(g * L, L)] = zvec
                return carry

            lax.fori_loop(g0 + 4 * n4, gr, z1, 0)

        @pl.when(val > 0.0)
        def _():
            def inner(g, carry):
                base = g * L
                bufs[b, j, pl.ds(base, L)] = bufs[b, j, pl.ds(base, L)] * val
                return carry

            lax.fori_loop(g0, gr, inner, 0)


def _sc_body(x_hbm, cv_hbm, o_hbm, cvv, bufs, lsem, ssem):
    cid = lax.axis_index("c")
    sid = lax.axis_index("s")
    wid = sid * NC + cid
    pltpu.sync_copy(cv_hbm, cvv)
    cv = cvv[pl.ds(0, L)][0]
    row0 = wid * RPW

    def load(c, b):
        pltpu.async_copy(x_hbm.at[pl.ds(row0 + c * CH, CH)],
                         bufs.at[b], lsem.at[b])

    def wait_load(b):
        pltpu.make_async_copy(x_hbm.at[pl.ds(0, CH)],
                              bufs.at[b], lsem.at[b]).wait()

    def store(c, b):
        pltpu.async_copy(bufs.at[b],
                         o_hbm.at[pl.ds(row0 + c * CH, CH)], ssem.at[b])

    def wait_store(b):
        pltpu.make_async_copy(bufs.at[b],
                              o_hbm.at[pl.ds(0, CH)], ssem.at[b]).wait()

    for c0 in range(PF):
        load(c0, c0 % NB)

    def outer(t, carry):
        c_base = t * NB
        for b in range(NB):
            c = c_base + b

            @pl.when(c + PF < NCHUNK)
            def _(c=c, b=b):
                @pl.when(c + PF - NB >= 0)
                def _():
                    wait_store((b + PF) % NB)
                load(c + PF, (b + PF) % NB)

            wait_load(b)
            for j in range(CH):
                _scale_row(bufs, b, j, row0 + c * CH + j, cv)
            store(c, b)
        return carry

    lax.fori_loop(0, NCHUNK // NB, outer, 0)

    for k in range(NB):
        wait_store((NCHUNK - NB + k) % NB)


def kernel(x, current_val):
    B, H, Sr, Sc = x.shape
    x2 = x.reshape(B * H * Sr, Sc)
    cvp = jnp.pad(current_val.astype(jnp.float32), (0, L - 1))
    mesh = plsc.VectorSubcoreMesh(core_axis_name="c", subcore_axis_name="s")
    k = functools.partial(
        pl.kernel,
        mesh=mesh,
        out_type=jax.ShapeDtypeStruct((ROWS, S), jnp.float32),
        scratch_types=[
            pltpu.VMEM((L,), jnp.float32),
            pltpu.VMEM((NB, CH, S), jnp.float32),
            pltpu.SemaphoreType.DMA((NB,)),
            pltpu.SemaphoreType.DMA((NB,)),
        ],
    )(_sc_body)
    out = k(x2, cvp)
    return out.reshape(B, H, Sr, Sc)


# speedup vs baseline: 6.8077x; 1.8509x over previous
import jax
import jax.numpy as jnp
from jax.experimental import pallas as pl
from jax.experimental.pallas import tpu as pltpu

S = 2048
ROWS_PER_BLOCK = 512


def _body(cv_ref, x_ref, o_ref):
    blk = pl.program_id(0)
    R, C = x_ref.shape
    cv = cv_ref[0]
    g = blk * R + jax.lax.broadcasted_iota(jnp.int32, (R, 1), 0)
    r = jax.lax.rem(g, S)
    i = jnp.minimum(r, S - 1 - r)
    val = jnp.clip((i.astype(jnp.float32) - 991.0 + 2048.0 * cv) * (1.0 / 32.0),
                   0.0, 1.0)
    left = i + jnp.where(r >= S // 2, 1, 0)
    width = (S - 1 - i) - left
    c = jax.lax.broadcasted_iota(jnp.int32, (R, C), 1)
    d = c - left
    cond = d.astype(jnp.uint32) <= width.astype(jnp.uint32)
    o_ref[...] = x_ref[...] * jnp.where(cond, val, 1.0)


def kernel(x, current_val):
    B, H, Sr, Sc = x.shape
    x2 = x.reshape(B * H * Sr, Sc)
    n_rows = x2.shape[0]
    grid = (n_rows // ROWS_PER_BLOCK,)
    out = pl.pallas_call(
        _body,
        grid=grid,
        in_specs=[
            pl.BlockSpec(memory_space=pltpu.SMEM),
            pl.BlockSpec((ROWS_PER_BLOCK, Sc), lambda b: (b, 0)),
        ],
        out_specs=pl.BlockSpec((ROWS_PER_BLOCK, Sc), lambda b: (b, 0)),
        out_shape=jax.ShapeDtypeStruct((n_rows, Sc), x.dtype),
    )(current_val, x2)
    return out.reshape(B, H, Sr, Sc)


# speedup vs baseline: 6.9186x; 1.0163x over previous
import jax
import jax.numpy as jnp
from jax.experimental import pallas as pl
from jax.experimental.pallas import tpu as pltpu

S = 2048
ROWS_PER_BLOCK = 1024


def _body(cv_ref, x_ref, o_ref):
    blk = pl.program_id(0)
    R, C = x_ref.shape
    cv = cv_ref[0]
    g = blk * R + jax.lax.broadcasted_iota(jnp.int32, (R, 1), 0)
    r = jax.lax.rem(g, S)
    i = jnp.minimum(r, S - 1 - r)
    val = jnp.clip((i.astype(jnp.float32) - 991.0 + 2048.0 * cv) * (1.0 / 32.0),
                   0.0, 1.0)
    left = i + jnp.where(r >= S // 2, 1, 0)
    width = (S - 1 - i) - left
    c = jax.lax.broadcasted_iota(jnp.int32, (R, C), 1)
    d = c - left
    cond = d.astype(jnp.uint32) <= width.astype(jnp.uint32)
    o_ref[...] = x_ref[...] * jnp.where(cond, val, 1.0)


def kernel(x, current_val):
    B, H, Sr, Sc = x.shape
    x2 = x.reshape(B * H * Sr, Sc)
    n_rows = x2.shape[0]
    grid = (n_rows // ROWS_PER_BLOCK,)
    out = pl.pallas_call(
        _body,
        grid=grid,
        in_specs=[
            pl.BlockSpec(memory_space=pltpu.SMEM),
            pl.BlockSpec((ROWS_PER_BLOCK, Sc), lambda b: (b, 0)),
        ],
        out_specs=pl.BlockSpec((ROWS_PER_BLOCK, Sc), lambda b: (b, 0)),
        out_shape=jax.ShapeDtypeStruct((n_rows, Sc), x.dtype),
    )(current_val, x2)
    return out.reshape(B, H, Sr, Sc)


# speedup vs baseline: 6.9680x; 1.0071x over previous
import jax
import jax.numpy as jnp
from jax.experimental import pallas as pl
from jax.experimental.pallas import tpu as pltpu

S = 2048
ROWS_PER_BLOCK = 1536


def _body(cv_ref, x_ref, o_ref):
    blk = pl.program_id(0)
    R, C = x_ref.shape
    cv = cv_ref[0]
    g = blk * R + jax.lax.broadcasted_iota(jnp.int32, (R, 1), 0)
    r = jax.lax.rem(g, S)
    i = jnp.minimum(r, S - 1 - r)
    val = jnp.clip((i.astype(jnp.float32) - 991.0 + 2048.0 * cv) * (1.0 / 32.0),
                   0.0, 1.0)
    left = i + jnp.where(r >= S // 2, 1, 0)
    width = (S - 1 - i) - left
    c = jax.lax.broadcasted_iota(jnp.int32, (R, C), 1)
    d = c - left
    cond = d.astype(jnp.uint32) <= width.astype(jnp.uint32)
    o_ref[...] = x_ref[...] * jnp.where(cond, val, 1.0)


def kernel(x, current_val):
    B, H, Sr, Sc = x.shape
    x2 = x.reshape(B * H * Sr, Sc)
    n_rows = x2.shape[0]
    grid = (n_rows // ROWS_PER_BLOCK,)
    out = pl.pallas_call(
        _body,
        grid=grid,
        in_specs=[
            pl.BlockSpec(memory_space=pltpu.SMEM),
            pl.BlockSpec((ROWS_PER_BLOCK, Sc), lambda b: (b, 0)),
        ],
        out_specs=pl.BlockSpec((ROWS_PER_BLOCK, Sc), lambda b: (b, 0)),
        out_shape=jax.ShapeDtypeStruct((n_rows, Sc), x.dtype),
    )(current_val, x2)
    return out.reshape(B, H, Sr, Sc)
